# baseline (speedup 1.0000x reference)
import jax
import jax.numpy as jnp
from jax.experimental import pallas as pl
from jax.experimental.pallas import tpu as pltpu

_N = 10000
_F = 128
_BM = 80


def _gc_kernel(x_ref, w_ref, b1_ref, e_ref, bias_ref, out_ref, xw_ref):
    i = pl.program_id(0)

    @pl.when(i == 0)
    def _():
        xw_ref[...] = (
            jnp.dot(x_ref[...], w_ref[...], preferred_element_type=jnp.float32)
            + b1_ref[...]
        )

    out_ref[...] = (
        jnp.dot(e_ref[...], xw_ref[...], preferred_element_type=jnp.float32)
        + bias_ref[...]
    )


def _graph_conv(X, W1, b1_2d, E2d, bias_2d, interpret=False):
    grid = (_N // _BM,)
    return pl.pallas_call(
        _gc_kernel,
        grid=grid,
        in_specs=[
            pl.BlockSpec((_N, _F), lambda i: (0, 0)),
            pl.BlockSpec((_F, _F), lambda i: (0, 0)),
            pl.BlockSpec((1, _F), lambda i: (0, 0)),
            pl.BlockSpec((_BM, _N), lambda i: (i, 0)),
            pl.BlockSpec((1, _F), lambda i: (0, 0)),
        ],
        out_specs=pl.BlockSpec((_BM, _F), lambda i: (i, 0)),
        out_shape=jax.ShapeDtypeStruct((_N, _F), jnp.float32),
        scratch_shapes=[pltpu.VMEM((_N, _F), jnp.float32)],
        compiler_params=pltpu.CompilerParams(
            dimension_semantics=("arbitrary",),
        ),
        interpret=interpret,
    )(X, W1, b1_2d, E2d, bias_2d)


def kernel(X, E, W1, b1, bias):
    E2d = E.reshape(_N, _N)
    b1_2d = b1.reshape(1, _F)
    bias_2d = bias.reshape(1, _F)
    return _graph_conv(X, W1, b1_2d, E2d, bias_2d)


# speedup vs baseline: 1.3707x; 1.3707x over previous
import jax
import jax.numpy as jnp
from jax.experimental import pallas as pl
from jax.experimental.pallas import tpu as pltpu

_N = 10000
_F = 128
_BM = 400


def _gc_kernel(x_ref, w_ref, b1_ref, e_ref, bias_ref, out_ref, xw_ref):
    i = pl.program_id(0)

    @pl.when(i == 0)
    def _():
        xw_ref[...] = (
            jnp.dot(x_ref[...], w_ref[...], preferred_element_type=jnp.float32)
            + b1_ref[...]
        )

    out_ref[...] = (
        jnp.dot(e_ref[...], xw_ref[...], preferred_element_type=jnp.float32)
        + bias_ref[...]
    )


def _graph_conv(X, W1, b1_2d, E2d, bias_2d, interpret=False):
    grid = (_N // _BM,)
    return pl.pallas_call(
        _gc_kernel,
        grid=grid,
        in_specs=[
            pl.BlockSpec((_N, _F), lambda i: (0, 0)),
            pl.BlockSpec((_F, _F), lambda i: (0, 0)),
            pl.BlockSpec((1, _F), lambda i: (0, 0)),
            pl.BlockSpec((_BM, _N), lambda i: (i, 0)),
            pl.BlockSpec((1, _F), lambda i: (0, 0)),
        ],
        out_specs=pl.BlockSpec((_BM, _F), lambda i: (i, 0)),
        out_shape=jax.ShapeDtypeStruct((_N, _F), jnp.float32),
        scratch_shapes=[pltpu.VMEM((_N, _F), jnp.float32)],
        compiler_params=pltpu.CompilerParams(
            dimension_semantics=("arbitrary",),
        ),
        interpret=interpret,
    )(X, W1, b1_2d, E2d, bias_2d)


def kernel(X, E, W1, b1, bias):
    E2d = E.reshape(_N, _N)
    b1_2d = b1.reshape(1, _F)
    bias_2d = bias.reshape(1, _F)
    return _graph_conv(X, W1, b1_2d, E2d, bias_2d)
